# baseline (speedup 1.0000x reference)
import jax
import jax.numpy as jnp
from jax import lax
from jax.experimental import pallas as pl
from jax.experimental.pallas import tpu as pltpu
from jax.experimental.pallas import tpu_sc as plsc

_IGNORE_INDEX = 10

_NC = 2
_NS = 16
_NW = _NC * _NS


def _make_scatter_kernel(N, C):
    rows_per_w = N // _NW
    assert N % _NW == 0 and rows_per_w % 128 == 0
    idx_rows = rows_per_w // 128

    mesh = plsc.VectorSubcoreMesh(core_axis_name="c", subcore_axis_name="s")

    @pl.kernel(
        mesh=mesh,
        out_type=(),
        scratch_types=[
            pltpu.VMEM((idx_rows, 128), jnp.int32),
            pltpu.VMEM((idx_rows, 128), jnp.float32),
            pltpu.SemaphoreType.DMA,
        ],
    )
    def kern(buf_hbm, idx_hbm, val_hbm, idx_v, val_v, ssem):
        wid = lax.axis_index("s") * _NC + lax.axis_index("c")
        pltpu.sync_copy(idx_hbm.at[wid], idx_v)
        pltpu.sync_copy(val_hbm.at[wid], val_v)
        scats = []
        for r in range(idx_rows):
            scats.append(
                pltpu.async_copy(val_v.at[r], buf_hbm.at[idx_v.at[r]], ssem))
        for s in scats:
            s.wait()

    return kern


def kernel(grad_output, input, target, total_weight):
    N, C = input.shape
    t = target.astype(jnp.int32)
    g = grad_output.astype(jnp.float32)
    idx = t * N + jnp.arange(N, dtype=jnp.int32)
    vals = jnp.where(t != _IGNORE_INDEX, -g, jnp.zeros_like(g))
    rows_per_w = N // _NW
    idx3 = idx.reshape(_NW, rows_per_w // 128, 128)
    val3 = vals.reshape(_NW, rows_per_w // 128, 128)
    buf = jax.new_ref(jnp.zeros((N * C,), jnp.float32))
    _make_scatter_kernel(N, C)(buf, idx3, val3)
    return buf[...].reshape(C, N).T
